# Initial kernel scaffold; baseline (speedup 1.0000x reference)
import jax
import jax.numpy as jnp
from jax.experimental import pallas as pl


def kernel(input, embedding_weight):
    raise NotImplementedError("write your pallas kernel here")


# speedup vs baseline: 1.1279x; 1.1279x over previous
import jax
import jax.numpy as jnp
from jax import lax
from jax.experimental import pallas as pl
from jax.experimental.pallas import tpu as pltpu
from jax.experimental.pallas import tpu_sc as plsc

NUM_E = 1000000
D = 32
BATCH = 16384
HIST = 50
NFLAT = BATCH * HIST
NC = 2
NS = 16
NW = NC * NS
PER_W = NFLAT // NW
CHUNK = 128
NCHUNK = PER_W // CHUNK


def _rtne16(u):
  odd = lax.shift_right_logical(u, 16) & 1
  return lax.shift_right_logical(u + 0x7FFF + odd, 16)


def _body(table_hbm, idx_hbm, out_hbm, idx_v, rows_v, obuf_v, gsem):
  wid = lax.axis_index("s") * NC + lax.axis_index("c")
  base = wid * PER_W
  pltpu.sync_copy(idx_hbm.at[wid], idx_v)

  col_even = 2 * lax.iota(jnp.int32, 16)
  col_odd = col_even + 1

  def chunk_body(j, _):
    pltpu.async_copy(table_hbm.at[idx_v.at[j]], rows_v, gsem).wait()

    @plsc.parallel_loop(0, CHUNK, unroll=4)
    def _row(r):
      rv = rows_v.at[r]
      a = plsc.bitcast(plsc.load_gather(rv, [col_even]), jnp.int32)
      b = plsc.bitcast(plsc.load_gather(rv, [col_odd]), jnp.int32)
      word = _rtne16(a) | lax.shift_left(_rtne16(b), 16)
      obuf_v[pl.ds(r * D, D)] = plsc.bitcast(word, jnp.bfloat16)

    pltpu.sync_copy(obuf_v, out_hbm.at[pl.ds((base + j * CHUNK) * D, CHUNK * D)])
    return 0

  lax.fori_loop(0, NCHUNK, chunk_body, 0)


def kernel(input, embedding_weight):
  idx = input.astype(jnp.int32).reshape(NW, NCHUNK, CHUNK)
  mesh = plsc.VectorSubcoreMesh(core_axis_name="c", subcore_axis_name="s")
  out = pl.kernel(
      _body,
      out_type=jax.ShapeDtypeStruct((NFLAT * D,), jnp.bfloat16),
      mesh=mesh,
      compiler_params=pltpu.CompilerParams(
          needs_layout_passes=False, use_tc_tiling_on_sc=False),
      scratch_types=[
          pltpu.VMEM((NCHUNK, CHUNK), jnp.int32),
          pltpu.VMEM((CHUNK, D), jnp.float32),
          pltpu.VMEM((CHUNK * D,), jnp.bfloat16),
          pltpu.SemaphoreType.DMA,
      ],
  )(embedding_weight, idx)
  return out.reshape(BATCH, HIST, D)


# speedup vs baseline: 1.3206x; 1.1708x over previous
import jax
import jax.numpy as jnp
from jax import lax
from jax.experimental import pallas as pl
from jax.experimental.pallas import tpu as pltpu
from jax.experimental.pallas import tpu_sc as plsc

NUM_E = 1000000
D = 32
BATCH = 16384
HIST = 50
NFLAT = BATCH * HIST
NC = 2
NS = 16
NW = NC * NS
PER_W = NFLAT // NW
CHUNK = 128
NCHUNK = PER_W // CHUNK
NBUF = 4


def _rtne16(u):
  odd = lax.shift_right_logical(u, 16) & 1
  return lax.shift_right_logical(u + 0x7FFF + odd, 16)


def _body(table_hbm, idx_hbm, out_hbm, idx_v, rows_v, obuf_v, gsem, wsem):
  wid = lax.axis_index("s") * NC + lax.axis_index("c")
  base = wid * PER_W
  pltpu.sync_copy(idx_hbm.at[wid], idx_v)

  col_even = 2 * lax.iota(jnp.int32, 16)
  col_odd = col_even + 1

  def gather_start(j, b):
    pltpu.async_copy(table_hbm.at[idx_v.at[j]], rows_v.at[b], gsem.at[b])

  def gather_wait(j, b):
    pltpu.make_async_copy(
        table_hbm.at[idx_v.at[j]], rows_v.at[b], gsem.at[b]).wait()

  def write_start(j, b):
    pltpu.async_copy(
        obuf_v.at[b], out_hbm.at[pl.ds((base + j * CHUNK) * D, CHUNK * D)],
        wsem.at[b])

  def write_wait(j, b):
    pltpu.make_async_copy(
        obuf_v.at[b], out_hbm.at[pl.ds((base + j * CHUNK) * D, CHUNK * D)],
        wsem.at[b]).wait()

  for b in range(NBUF):
    gather_start(b, b)

  def outer(j0, _):
    for b in range(NBUF):
      j = j0 * NBUF + b
      gather_wait(j, b)

      @pl.when(j >= NBUF)
      def _():
        write_wait(j - NBUF, b)

      rb = rows_v.at[b]
      ob = obuf_v.at[b]

      @plsc.parallel_loop(0, CHUNK, unroll=4)
      def _row(r):
        rv = rb.at[r]
        a = plsc.bitcast(plsc.load_gather(rv, [col_even]), jnp.int32)
        c = plsc.bitcast(plsc.load_gather(rv, [col_odd]), jnp.int32)
        word = _rtne16(a) | lax.shift_left(_rtne16(c), 16)
        ob[pl.ds(r * D, D)] = plsc.bitcast(word, jnp.bfloat16)

      write_start(j, b)

      @pl.when(j + NBUF < NCHUNK)
      def _():
        gather_start(j + NBUF, b)

    return 0

  lax.fori_loop(0, NCHUNK // NBUF, outer, 0)
  for b in range(NBUF):
    write_wait(NCHUNK - NBUF + b, b)


def kernel(input, embedding_weight):
  idx = input.astype(jnp.int32).reshape(NW, NCHUNK, CHUNK)
  mesh = plsc.VectorSubcoreMesh(core_axis_name="c", subcore_axis_name="s")
  out = pl.kernel(
      _body,
      out_type=jax.ShapeDtypeStruct((NFLAT * D,), jnp.bfloat16),
      mesh=mesh,
      compiler_params=pltpu.CompilerParams(
          needs_layout_passes=False, use_tc_tiling_on_sc=False),
      scratch_types=[
          pltpu.VMEM((NCHUNK, CHUNK), jnp.int32),
          pltpu.VMEM((NBUF, CHUNK, D), jnp.float32),
          pltpu.VMEM((NBUF, CHUNK * D), jnp.bfloat16),
          pltpu.SemaphoreType.DMA((NBUF,)),
          pltpu.SemaphoreType.DMA((NBUF,)),
      ],
  )(embedding_weight, idx)
  return out.reshape(BATCH, HIST, D)
